# baseline (speedup 1.0000x reference)
import functools

import jax
import jax.numpy as jnp
from jax import lax
from jax.experimental import pallas as pl
from jax.experimental.pallas import tpu as pltpu
from jax.experimental.pallas import tpu_sc as plsc

_NC = 2
_NS = 16
_NW = _NC * _NS
_CBS = 128
_CBW = 64
_UF = 8


def _make_qtable(B, F, Fh, f_base, add_prev):
    ns_chunks = B // (_NW * _CBS)
    nw_per_s = _CBS // _CBW
    nw_chunks = ns_chunks * nw_per_s
    b_per_w = ns_chunks * _CBS

    mesh = plsc.VectorSubcoreMesh(core_axis_name="c", subcore_axis_name="s")

    def qtable(*refs):
        if add_prev:
            (state_hbm, action_hbm, table_hbm, prev_hbm, out_hbm,
             idx_v, w0, w1, s0, s1, obuf, sw0, sw1, ss0, ss1) = refs
        else:
            (state_hbm, action_hbm, table_hbm, out_hbm,
             idx_v, w0, w1, s0, s1, obuf, sw0, sw1, ss0, ss1) = refs
        wid = lax.axis_index("s") * _NC + lax.axis_index("c")
        base = wid * b_per_w
        pltpu.sync_copy(action_hbm.at[pl.ds(base, b_per_w)], idx_v)
        if add_prev:
            pltpu.sync_copy(prev_hbm.at[pl.ds(base, b_per_w)], obuf)
        wbufs = ((w0, sw0), (w1, sw1))
        sbufs = ((s0, ss0), (s1, ss1))

        pending_w = {}
        pending_s = {}

        def start_w(cw):
            wb, sem = wbufs[cw % 2]
            h = pltpu.make_async_copy(
                table_hbm.at[idx_v.at[pl.ds(cw * _CBW, _CBW)]], wb, sem)
            h.start()
            pending_w[cw] = h

        def start_s(cs):
            sb, sem = sbufs[cs % 2]
            h = pltpu.make_async_copy(
                state_hbm.at[pl.ds(f_base, Fh),
                             pl.ds(base + cs * _CBS, _CBS)], sb, sem)
            h.start()
            pending_s[cs] = h

        lane = lax.broadcasted_iota(jnp.int32, (16,), 0)
        zf = jnp.zeros((16,), jnp.float32)
        zi = jnp.zeros((16,), jnp.int32)

        start_s(0)
        if ns_chunks > 1:
            start_s(1)
        start_w(0)
        if nw_chunks > 1:
            start_w(1)

        for cs in range(ns_chunks):
            pending_s.pop(cs).wait()
            sb = sbufs[cs % 2][0]
            for h in range(nw_per_s):
                cw = cs * nw_per_s + h
                pending_w.pop(cw).wait()
                wb = wbufs[cw % 2][0]
                for g in range(_CBW // 16):
                    rows = lane + (g * 16)
                    col0 = h * _CBW + g * 16

                    def fbody(f0, accs, rows=rows, wb=wb, sb=sb, col0=col0):
                        a0, a1 = accs
                        for u in range(0, _UF, 2):
                            col = zi + (f0 + u)
                            w = plsc.load_gather(wb, [rows, col])
                            s = sb[f0 + u, pl.ds(col0, 16)]
                            a0 = a0 + w * s
                            col = zi + (f0 + u + 1)
                            w = plsc.load_gather(wb, [rows, col])
                            s = sb[f0 + u + 1, pl.ds(col0, 16)]
                            a1 = a1 + w * s
                        return (a0, a1)

                    accs = plsc.parallel_loop(
                        0, Fh, step=_UF, carry=(zf, zf))(fbody)
                    acc = accs[0] + accs[1]
                    o0 = cw * _CBW + g * 16
                    if add_prev:
                        obuf[pl.ds(o0, 16)] = acc + obuf[pl.ds(o0, 16)]
                    else:
                        obuf[pl.ds(o0, 16)] = acc
                if cw + 2 < nw_chunks:
                    start_w(cw + 2)
            if cs + 2 < ns_chunks:
                start_s(cs + 2)
        pltpu.sync_copy(obuf, out_hbm.at[pl.ds(base, b_per_w)])

    return functools.partial(
        pl.kernel,
        mesh=mesh,
        compiler_params=pltpu.CompilerParams(needs_layout_passes=False),
        out_type=jax.ShapeDtypeStruct((B,), jnp.float32),
        scratch_types=[
            pltpu.VMEM((b_per_w,), jnp.int32),
            pltpu.VMEM((_CBW, Fh), jnp.float32),
            pltpu.VMEM((_CBW, Fh), jnp.float32),
            pltpu.VMEM((Fh, _CBS), jnp.float32),
            pltpu.VMEM((Fh, _CBS), jnp.float32),
            pltpu.VMEM((b_per_w,), jnp.float32),
            pltpu.SemaphoreType.DMA,
            pltpu.SemaphoreType.DMA,
            pltpu.SemaphoreType.DMA,
            pltpu.SemaphoreType.DMA,
        ],
    )(qtable)


def kernel(state, action, weights):
    B, F1, F2 = state.shape
    F = F1 * F2
    V = weights.shape[0]
    Fh = F // 2
    assert B % (_NW * _CBS) == 0 and Fh % _UF == 0 and Fh % 128 == 0

    stateT = state.transpose(1, 2, 0).reshape(F, B)
    action32 = action.astype(jnp.int32)
    table_a = weights[:, : F1 // 2, :].reshape(V, Fh)
    table_b = weights[:, F1 // 2:, :].reshape(V, Fh)

    pass_a = _make_qtable(B, F, Fh, 0, add_prev=False)
    pass_b = _make_qtable(B, F, Fh, Fh, add_prev=True)

    part = pass_a(stateT, action32, table_a)
    return pass_b(stateT, action32, table_b, part)
